# baseline (speedup 1.0000x reference)
import functools

import jax
import jax.numpy as jnp
from jax import lax
from jax.experimental import pallas as pl
from jax.experimental.pallas import tpu as pltpu
from jax.experimental.pallas import tpu_sc as plsc

_F = 26
_V = 100000
_E = 16
_B = 4096
_BCONST = 0.99

_NC = 2
_NS = 16
_NW = _NC * _NS
_BPW = _B // _NW
_JPW = _BPW * _F
_NG = _JPW // _BPW


def _fm_body(xi_h, xv_h, fw_h, sw_h, out_h,
             xi_v, xv_v, idx_v, fw_v, rows_v, out_v, sem):
    wid = lax.axis_index("s") * _NC + lax.axis_index("c")
    jbase = wid * _JPW
    pltpu.sync_copy(xi_h.at[pl.ds(jbase, _JPW)], xi_v)
    pltpu.sync_copy(xv_h.at[pl.ds(jbase, _JPW)], xv_v)

    lane = lax.iota(jnp.int32, 16)

    def idx_body(g, carry):
        for t in range(8):
            k = g * 8 + t
            xi = xi_v[pl.ds(k * 16, 16)]
            fld = (k * 16 + lane) % _F
            idx_v[g, pl.ds(t * 16, 16)] = xi + fld * _V
        return carry
    lax.fori_loop(0, _NG, idx_body, 0)

    copies = []
    for g in range(_NG):
        dst = pl.ds(g * _BPW, _BPW)
        copies.append(pltpu.async_copy(sw_h.at[idx_v.at[g]], rows_v.at[dst], sem))
        copies.append(pltpu.async_copy(fw_h.at[idx_v.at[g]], fw_v.at[dst], sem))
    for c in copies:
        c.wait()

    mB = lane < (_F - 16)

    def b_body(b, carry):
        jb = b * _F
        la = jb + lane
        fwA = plsc.load_gather(fw_v, [la])
        xvA = plsc.load_gather(xv_v, [la])
        fwB = plsc.load_gather(fw_v, [la + 16], mask=mB)
        xvB = plsc.load_gather(xv_v, [la + 16], mask=mB)
        fo = fwA * xvA + jnp.where(mB, fwB * xvB, 0.0)
        acc = jnp.zeros((16,), jnp.float32)
        sq = jnp.zeros((16,), jnp.float32)
        for i in range(_F):
            xb = plsc.load_gather(xv_v, [jnp.full((16,), jb + i, jnp.int32)])
            v = rows_v[jb + i, :] * xb
            acc = acc + v
            sq = sq + v * v
        tv = fo + 0.5 * (acc * acc - sq)
        tv = tv + jnp.where(lane == 0, jnp.float32(_BCONST), 0.0)
        cum = plsc.cumsum(tv)
        plsc.store_scatter(out_v, [jnp.full((16,), b, jnp.int32)], cum,
                           mask=lane == 15)
        return carry
    lax.fori_loop(0, _BPW, b_body, 0)

    pltpu.sync_copy(out_v, out_h.at[pl.ds(wid * _BPW, _BPW)])


def _fm_call(xi, xv, fw, sw, interpret=False):
    mesh = plsc.VectorSubcoreMesh(core_axis_name="c", subcore_axis_name="s")
    return pl.kernel(
        _fm_body,
        out_type=jax.ShapeDtypeStruct((_B,), jnp.float32),
        mesh=mesh,
        scratch_types=[
            pltpu.VMEM((_JPW,), jnp.int32),
            pltpu.VMEM((_JPW,), jnp.float32),
            pltpu.VMEM((_NG, _BPW), jnp.int32),
            pltpu.VMEM((_JPW,), jnp.float32),
            pltpu.VMEM((_JPW, _E), jnp.float32),
            pltpu.VMEM((_BPW,), jnp.float32),
            pltpu.SemaphoreType.DMA,
        ],
        compiler_params=pltpu.CompilerParams(needs_layout_passes=False,
                                             use_tc_tiling_on_sc=False),
        interpret=interpret,
    )(xi, xv, fw, sw)


@jax.jit
def kernel(Xi, Xv, first_w, second_w):
    xi = Xi.reshape(_B * _F).astype(jnp.int32)
    xv = Xv.reshape(_B * _F)
    fw = first_w.reshape(_F * _V)
    sw = second_w.reshape(_F * _V, _E)
    return _fm_call(xi, xv, fw, sw)
